# baseline (speedup 1.0000x reference)
import functools

import jax
import jax.numpy as jnp
from jax import lax
from jax.experimental import pallas as pl
from jax.experimental.pallas import tpu as pltpu
from jax.experimental.pallas import tpu_sc as plsc

_B = 256
_T = 1000
_F = 4 * 64 * 64
_L = 16
_NW = _B // _L
_NC = 2


def _sc_gather_body(t_hbm, a_hbm, b_hbm, c1_hbm, c2_hbm,
                    idx_v, o1_v, o2_v, sem1, sem2):
    wid = lax.axis_index("s")

    @pl.when(wid < _NW)
    def _():
        sl = pl.ds(wid * _L, _L)
        pltpu.sync_copy(t_hbm.at[sl], idx_v)
        g1 = pltpu.async_copy(a_hbm.at[idx_v], o1_v, sem1)
        g2 = pltpu.async_copy(b_hbm.at[idx_v], o2_v, sem2)
        g1.wait()
        g2.wait()
        w1 = pltpu.async_copy(o1_v, c1_hbm.at[sl], sem1)
        w2 = pltpu.async_copy(o2_v, c2_hbm.at[sl], sem2)
        w1.wait()
        w2.wait()


_sc_gather = functools.partial(
    pl.kernel,
    mesh=plsc.VectorSubcoreMesh(core_axis_name="c", subcore_axis_name="s", num_cores=1),
    out_type=(
        jax.ShapeDtypeStruct((_B,), jnp.float32),
        jax.ShapeDtypeStruct((_B,), jnp.float32),
    ),
    scratch_types=[
        pltpu.VMEM((_L,), jnp.int32),
        pltpu.VMEM((_L,), jnp.float32),
        pltpu.VMEM((_L,), jnp.float32),
        pltpu.SemaphoreType.DMA,
        pltpu.SemaphoreType.DMA,
    ],
)(_sc_gather_body)


_ROWS = 1024
_GRID = _F // _ROWS


def _combine_body(c1_ref, c2_ref, x_ref, n_ref, o_ref):
    o_ref[...] = c1_ref[...] * x_ref[...] + c2_ref[...] * n_ref[...]


def _combine(c1, c2, x, n):
    return pl.pallas_call(
        _combine_body,
        grid=(_GRID,),
        in_specs=[
            pl.BlockSpec((1, _B), lambda i: (0, 0)),
            pl.BlockSpec((1, _B), lambda i: (0, 0)),
            pl.BlockSpec((_ROWS, _B), lambda i: (i, 0)),
            pl.BlockSpec((_ROWS, _B), lambda i: (i, 0)),
        ],
        out_specs=pl.BlockSpec((_ROWS, _B), lambda i: (i, 0)),
        out_shape=jax.ShapeDtypeStruct((_F, _B), jnp.float32),
    )(c1, c2, x, n)


def kernel(img, noise, t, sqrt_a_bar, sqrt_one_minus_a_bar):
    c1, c2 = _sc_gather(t, sqrt_a_bar, sqrt_one_minus_a_bar)
    xt = img.transpose(1, 2, 3, 0).reshape(_F, _B)
    nt = noise.transpose(1, 2, 3, 0).reshape(_F, _B)
    out = _combine(c1.reshape(1, _B), c2.reshape(1, _B), xt, nt)
    return out.reshape(img.shape[1:] + (_B,)).transpose(3, 0, 1, 2)


# speedup vs baseline: 1.0850x; 1.0850x over previous
import functools

import jax
import jax.numpy as jnp
from jax import lax
from jax.experimental import pallas as pl
from jax.experimental.pallas import tpu as pltpu
from jax.experimental.pallas import tpu_sc as plsc

_B = 256
_T = 1000
_F = 4 * 64 * 64
_L = 16
_NW = _B // _L
_NC = 2


def _sc_gather_body(t_hbm, a_hbm, b_hbm, c1_hbm, c2_hbm,
                    idx_v, o1_v, o2_v, sem1, sem2):
    wid = lax.axis_index("s")

    @pl.when(wid < _NW)
    def _():
        sl = pl.ds(wid * _L, _L)
        pltpu.sync_copy(t_hbm.at[sl], idx_v)
        g1 = pltpu.async_copy(a_hbm.at[idx_v], o1_v, sem1)
        g2 = pltpu.async_copy(b_hbm.at[idx_v], o2_v, sem2)
        g1.wait()
        g2.wait()
        w1 = pltpu.async_copy(o1_v, c1_hbm.at[sl], sem1)
        w2 = pltpu.async_copy(o2_v, c2_hbm.at[sl], sem2)
        w1.wait()
        w2.wait()


_sc_gather = functools.partial(
    pl.kernel,
    mesh=plsc.VectorSubcoreMesh(core_axis_name="c", subcore_axis_name="s", num_cores=1),
    out_type=(
        jax.ShapeDtypeStruct((_B,), jnp.float32),
        jax.ShapeDtypeStruct((_B,), jnp.float32),
    ),
    scratch_types=[
        pltpu.VMEM((_L,), jnp.int32),
        pltpu.VMEM((_L,), jnp.float32),
        pltpu.VMEM((_L,), jnp.float32),
        pltpu.SemaphoreType.DMA,
        pltpu.SemaphoreType.DMA,
    ],
)(_sc_gather_body)


_ROWS = 4096
_GRID = _F // _ROWS


def _combine_body(c1_ref, c2_ref, x_ref, n_ref, o_ref):
    o_ref[...] = c1_ref[...] * x_ref[...] + c2_ref[...] * n_ref[...]


def _combine(c1, c2, x, n):
    return pl.pallas_call(
        _combine_body,
        grid=(_GRID,),
        in_specs=[
            pl.BlockSpec((1, _B), lambda i: (0, 0)),
            pl.BlockSpec((1, _B), lambda i: (0, 0)),
            pl.BlockSpec((_ROWS, _B), lambda i: (i, 0)),
            pl.BlockSpec((_ROWS, _B), lambda i: (i, 0)),
        ],
        out_specs=pl.BlockSpec((_ROWS, _B), lambda i: (i, 0)),
        out_shape=jax.ShapeDtypeStruct((_F, _B), jnp.float32),
    )(c1, c2, x, n)


def kernel(img, noise, t, sqrt_a_bar, sqrt_one_minus_a_bar):
    c1, c2 = _sc_gather(t, sqrt_a_bar, sqrt_one_minus_a_bar)
    xt = img.transpose(1, 2, 3, 0).reshape(_F, _B)
    nt = noise.transpose(1, 2, 3, 0).reshape(_F, _B)
    out = _combine(c1.reshape(1, _B), c2.reshape(1, _B), xt, nt)
    return out.reshape(img.shape[1:] + (_B,)).transpose(3, 0, 1, 2)


# speedup vs baseline: 1.1306x; 1.0420x over previous
import functools

import jax
import jax.numpy as jnp
from jax import lax
from jax.experimental import pallas as pl
from jax.experimental.pallas import tpu as pltpu
from jax.experimental.pallas import tpu_sc as plsc

_B = 256
_T = 1000
_F = 4 * 64 * 64
_L = 16
_NW = _B // _L
_NC = 2


def _sc_gather_body(t_hbm, a_hbm, b_hbm, c1_hbm, c2_hbm,
                    idx_v, o1_v, o2_v, sem1, sem2):
    wid = lax.axis_index("s")

    @pl.when(wid < _NW)
    def _():
        sl = pl.ds(wid * _L, _L)
        pltpu.sync_copy(t_hbm.at[sl], idx_v)
        g1 = pltpu.async_copy(a_hbm.at[idx_v], o1_v, sem1)
        g2 = pltpu.async_copy(b_hbm.at[idx_v], o2_v, sem2)
        g1.wait()
        g2.wait()
        w1 = pltpu.async_copy(o1_v, c1_hbm.at[sl], sem1)
        w2 = pltpu.async_copy(o2_v, c2_hbm.at[sl], sem2)
        w1.wait()
        w2.wait()


_sc_gather = functools.partial(
    pl.kernel,
    mesh=plsc.VectorSubcoreMesh(core_axis_name="c", subcore_axis_name="s", num_cores=1),
    out_type=(
        jax.ShapeDtypeStruct((_B,), jnp.float32),
        jax.ShapeDtypeStruct((_B,), jnp.float32),
    ),
    scratch_types=[
        pltpu.VMEM((_L,), jnp.int32),
        pltpu.VMEM((_L,), jnp.float32),
        pltpu.VMEM((_L,), jnp.float32),
        pltpu.SemaphoreType.DMA,
        pltpu.SemaphoreType.DMA,
    ],
)(_sc_gather_body)


_ROWS = 8192
_GRID = _F // _ROWS


def _combine_body(c1_ref, c2_ref, x_ref, n_ref, o_ref):
    o_ref[...] = c1_ref[...] * x_ref[...] + c2_ref[...] * n_ref[...]


def _combine(c1, c2, x, n):
    return pl.pallas_call(
        _combine_body,
        grid=(_GRID,),
        in_specs=[
            pl.BlockSpec((1, _B), lambda i: (0, 0)),
            pl.BlockSpec((1, _B), lambda i: (0, 0)),
            pl.BlockSpec((_ROWS, _B), lambda i: (i, 0)),
            pl.BlockSpec((_ROWS, _B), lambda i: (i, 0)),
        ],
        out_specs=pl.BlockSpec((_ROWS, _B), lambda i: (i, 0)),
        out_shape=jax.ShapeDtypeStruct((_F, _B), jnp.float32),
    )(c1, c2, x, n)


def kernel(img, noise, t, sqrt_a_bar, sqrt_one_minus_a_bar):
    c1, c2 = _sc_gather(t, sqrt_a_bar, sqrt_one_minus_a_bar)
    xt = img.transpose(1, 2, 3, 0).reshape(_F, _B)
    nt = noise.transpose(1, 2, 3, 0).reshape(_F, _B)
    out = _combine(c1.reshape(1, _B), c2.reshape(1, _B), xt, nt)
    return out.reshape(img.shape[1:] + (_B,)).transpose(3, 0, 1, 2)
